# baseline (speedup 1.0000x reference)
import functools

import jax
import jax.numpy as jnp
from jax import lax
from jax.experimental import pallas as pl
from jax.experimental.pallas import tpu as pltpu
from jax.experimental.pallas import tpu_sc as plsc

NUM_CORES = 2
NUM_SUBCORES = 16
NUM_WORKERS = NUM_CORES * NUM_SUBCORES
NB = 4


def kernel(tokens, table, register_tokens):
    B, S = tokens.shape
    V, D = table.shape
    R = register_tokens.shape[0]
    T = R + S
    D8 = D // 8
    CB = B // 128
    NGRP = S // NB

    tokens_t = tokens.T.astype(jnp.int32)
    regs5 = jnp.broadcast_to(
        register_tokens.reshape(R, D8, 8, 1), (R, D8, 8, 128)
    )
    mesh = plsc.VectorSubcoreMesh(core_axis_name="c", subcore_axis_name="s")

    @functools.partial(
        pl.kernel,
        mesh=mesh,
        out_type=jax.ShapeDtypeStruct((T, D8, CB, 8, 128), jnp.float32),
        compiler_params=pltpu.CompilerParams(
            use_tc_tiling_on_sc=False, needs_layout_passes=False
        ),
        scratch_types=[
            pltpu.VMEM((S, 128), jnp.int32),
            pltpu.VMEM((NB, 128, D), jnp.float32),
            pltpu.VMEM((NB, D8, 8, 128), jnp.float32),
            pltpu.SemaphoreType.DMA((NB,)),
            pltpu.SemaphoreType.DMA((NB,)),
        ],
    )
    def emb(tok_hbm, table_hbm, regs_hbm, out_hbm, idx_all, emb_v, trn_v, gsem, tsem):
        wid = lax.axis_index("s") * NUM_CORES + lax.axis_index("c")
        c = wid
        pltpu.sync_copy(tok_hbm.at[:, pl.ds(c * 128, 128)], idx_all)
        iota = lax.iota(jnp.int32, 16)

        def gather_copy(slot, s):
            return pltpu.make_async_copy(
                table_hbm.at[idx_all.at[s]], emb_v.at[slot], gsem.at[slot]
            )

        def fire_out(slot, u):
            for r in range(D8):
                pltpu.async_copy(
                    trn_v.at[slot, r], out_hbm.at[u, r, c], tsem.at[slot]
                )

        def wait_out(slot, u):
            for r in range(D8):
                pltpu.make_async_copy(
                    trn_v.at[slot, r], out_hbm.at[u, r, c], tsem.at[slot]
                ).wait()

        for t in range(R):
            if t >= NB:
                wait_out(t % NB, t - NB)
            pltpu.sync_copy(regs_hbm.at[t], trn_v.at[t % NB])
            fire_out(t % NB, t)

        for b in range(NB):
            gather_copy(b, b).start()

        def body(g, _):
            for b in range(NB):
                s = g * NB + b
                u = R + s
                gather_copy(b, s).wait()
                prev_done = lax.select(g > 0, u - NB, R - NB + b)
                wait_out(b, prev_done)

                def col(d, _):
                    d2 = d // 8
                    d8 = d % 8
                    dvec = jnp.full((16,), 0, jnp.int32) + d
                    for k in range(8):
                        v = plsc.load_gather(
                            emb_v,
                            [
                                jnp.full((16,), b, jnp.int32),
                                iota + (16 * k),
                                dvec,
                            ],
                        )
                        trn_v[b, d2, d8, pl.ds(16 * k, 16)] = v
                    return ()

                lax.fori_loop(0, D, col, ())
                fire_out(b, u)

                @pl.when(g < NGRP - 1)
                def _():
                    gather_copy(b, s + NB).start()

            return ()

        lax.fori_loop(0, NGRP, body, ())

        for b in range(NB):
            wait_out(b, T - NB + b)

    out5 = emb(tokens_t, table, regs5)
    return jnp.transpose(out5, (2, 4, 0, 1, 3)).reshape(B, T, D)


# speedup vs baseline: 1.0015x; 1.0015x over previous
import functools

import jax
import jax.numpy as jnp
from jax import lax
from jax.experimental import pallas as pl
from jax.experimental.pallas import tpu as pltpu
from jax.experimental.pallas import tpu_sc as plsc

NUM_CORES = 2
NUM_SUBCORES = 16
NUM_WORKERS = NUM_CORES * NUM_SUBCORES
NB = 4


def kernel(tokens, table, register_tokens):
    B, S = tokens.shape
    V, D = table.shape
    R = register_tokens.shape[0]
    T = R + S
    D8 = D // 8
    CB = B // 128
    NGRP = S // NB

    tokens_t = tokens.T.astype(jnp.int32)
    regs5 = jnp.broadcast_to(
        register_tokens.reshape(R, D8, 8, 1), (R, D8, 8, 128)
    )
    mesh = plsc.VectorSubcoreMesh(core_axis_name="c", subcore_axis_name="s")

    @functools.partial(
        pl.kernel,
        mesh=mesh,
        out_type=jax.ShapeDtypeStruct((T, D8, CB, 8, 128), jnp.float32),
        compiler_params=pltpu.CompilerParams(
            use_tc_tiling_on_sc=False, needs_layout_passes=False
        ),
        scratch_types=[
            pltpu.VMEM((S, 128), jnp.int32),
            pltpu.VMEM((NB, 128, D), jnp.float32),
            pltpu.VMEM((NB, D8, 8, 128), jnp.float32),
            pltpu.SemaphoreType.DMA((NB,)),
            pltpu.SemaphoreType.DMA((NB,)),
        ],
    )
    def emb(tok_hbm, table_hbm, regs_hbm, out_hbm, idx_all, emb_v, trn_v, gsem, tsem):
        wid = lax.axis_index("s") * NUM_CORES + lax.axis_index("c")
        c = wid
        pltpu.sync_copy(tok_hbm.at[:, pl.ds(c * 128, 128)], idx_all)
        iota = lax.iota(jnp.int32, 16)

        def gather_copy(slot, s):
            return pltpu.make_async_copy(
                table_hbm.at[idx_all.at[s]], emb_v.at[slot], gsem.at[slot]
            )

        def fire_out(slot, u):
            pltpu.async_copy(trn_v.at[slot], out_hbm.at[u, :, c], tsem.at[slot])

        def wait_out(slot, u):
            pltpu.make_async_copy(
                trn_v.at[slot], out_hbm.at[u, :, c], tsem.at[slot]
            ).wait()

        for t in range(R):
            if t >= NB:
                wait_out(t % NB, t - NB)
            pltpu.sync_copy(regs_hbm.at[t], trn_v.at[t % NB])
            fire_out(t % NB, t)

        for b in range(NB):
            gather_copy(b, b).start()

        def body(g, _):
            for b in range(NB):
                s = g * NB + b
                u = R + s
                gather_copy(b, s).wait()
                prev_done = lax.select(g > 0, u - NB, R - NB + b)
                wait_out(b, prev_done)

                def col(d, _):
                    d2 = d // 8
                    d8 = d % 8
                    dvec = jnp.full((16,), 0, jnp.int32) + d
                    for k in range(8):
                        v = plsc.load_gather(
                            emb_v,
                            [
                                jnp.full((16,), b, jnp.int32),
                                iota + (16 * k),
                                dvec,
                            ],
                        )
                        trn_v[b, d2, d8, pl.ds(16 * k, 16)] = v
                    return ()

                lax.fori_loop(0, D, col, ())
                fire_out(b, u)

                @pl.when(g < NGRP - 1)
                def _():
                    gather_copy(b, s + NB).start()

            return ()

        lax.fori_loop(0, NGRP, body, ())

        for b in range(NB):
            wait_out(b, T - NB + b)

    out5 = emb(tokens_t, table, regs5)
    return jnp.transpose(out5, (2, 4, 0, 1, 3)).reshape(B, T, D)


# speedup vs baseline: 1.2020x; 1.2002x over previous
import functools

import jax
import jax.numpy as jnp
from jax import lax
from jax.experimental import pallas as pl
from jax.experimental.pallas import tpu as pltpu
from jax.experimental.pallas import tpu_sc as plsc

NUM_CORES = 2
NUM_SUBCORES = 16
NUM_WORKERS = NUM_CORES * NUM_SUBCORES
NB = 4


def kernel(tokens, table, register_tokens):
    B, S = tokens.shape
    V, D = table.shape
    R = register_tokens.shape[0]
    T = R + S
    D8 = D // 8
    CB = B // 128
    NGRP = S // NB

    tokens_t = tokens.T.astype(jnp.int32)
    regs5 = jnp.broadcast_to(
        register_tokens.reshape(R, D8, 8, 1), (R, D8, 8, 128)
    )
    mesh = plsc.VectorSubcoreMesh(core_axis_name="c", subcore_axis_name="s")

    @functools.partial(
        pl.kernel,
        mesh=mesh,
        out_type=jax.ShapeDtypeStruct((T, D8, CB, 8, 128), jnp.float32),
        compiler_params=pltpu.CompilerParams(
            use_tc_tiling_on_sc=False, needs_layout_passes=False
        ),
        scratch_types=[
            pltpu.VMEM((S, 128), jnp.int32),
            pltpu.VMEM((NB, 128, D), jnp.float32),
            pltpu.VMEM((NB, D8, 8, 128), jnp.float32),
            pltpu.SemaphoreType.DMA((NB,)),
            pltpu.SemaphoreType.DMA((NB,)),
        ],
    )
    def emb(tok_hbm, table_hbm, regs_hbm, out_hbm, idx_all, emb_v, trn_v, gsem, tsem):
        wid = lax.axis_index("s") * NUM_CORES + lax.axis_index("c")
        c = wid
        pltpu.sync_copy(tok_hbm.at[:, pl.ds(c * 128, 128)], idx_all)
        iota = lax.iota(jnp.int32, 16)

        def gather_copy(slot, s):
            return pltpu.make_async_copy(
                table_hbm.at[idx_all.at[s]], emb_v.at[slot], gsem.at[slot]
            )

        def fire_out(slot, u):
            pltpu.async_copy(trn_v.at[slot], out_hbm.at[u, :, c], tsem.at[slot])

        def wait_out(slot, u):
            pltpu.make_async_copy(
                trn_v.at[slot], out_hbm.at[u, :, c], tsem.at[slot]
            ).wait()

        for t in range(R):
            if t >= NB:
                wait_out(t % NB, t - NB)
            pltpu.sync_copy(regs_hbm.at[t], trn_v.at[t % NB])
            fire_out(t % NB, t)

        for b in range(NB):
            gather_copy(b, b).start()

        def body(g, _):
            for b in range(NB):
                s = g * NB + b
                u = R + s
                gather_copy(b, s).wait()
                prev_done = lax.select(g > 0, u - NB, R - NB + b)
                wait_out(b, prev_done)

                @plsc.parallel_loop(0, D, unroll=2)
                def col(d):
                    d2 = d // 8
                    d8 = d % 8
                    dvec = jnp.full((16,), 0, jnp.int32) + d
                    vs = [
                        plsc.load_gather(
                            emb_v,
                            [
                                jnp.full((16,), b, jnp.int32),
                                iota + (16 * k),
                                dvec,
                            ],
                        )
                        for k in range(8)
                    ]
                    for k in range(8):
                        trn_v[b, d2, d8, pl.ds(16 * k, 16)] = vs[k]
                fire_out(b, u)

                @pl.when(g < NGRP - 1)
                def _():
                    gather_copy(b, s + NB).start()

            return ()

        lax.fori_loop(0, NGRP, body, ())

        for b in range(NB):
            wait_out(b, T - NB + b)

    out5 = emb(tokens_t, table, regs5)
    return jnp.transpose(out5, (2, 4, 0, 1, 3)).reshape(B, T, D)


# speedup vs baseline: 2.2806x; 1.8972x over previous
import functools

import jax
import jax.numpy as jnp
from jax import lax
from jax.experimental import pallas as pl
from jax.experimental.pallas import tpu as pltpu
from jax.experimental.pallas import tpu_sc as plsc

NUM_CORES = 2
NUM_SUBCORES = 16
NUM_WORKERS = NUM_CORES * NUM_SUBCORES
NB = 4


def kernel(tokens, table, register_tokens):
    B, S = tokens.shape
    V, D = table.shape
    R = register_tokens.shape[0]
    T = R + S
    D8 = D // 8
    CB = B // 128
    NGRP = S // NB

    tokens_t = tokens.T.astype(jnp.int32)
    regs5 = jnp.broadcast_to(
        register_tokens.reshape(R, D8, 8, 1), (R, D8, 8, 128)
    )
    mesh = plsc.VectorSubcoreMesh(core_axis_name="c", subcore_axis_name="s")

    @functools.partial(
        pl.kernel,
        mesh=mesh,
        out_type=jax.ShapeDtypeStruct((T, D8, CB, 8, 128), jnp.float32),
        compiler_params=pltpu.CompilerParams(
            use_tc_tiling_on_sc=False, needs_layout_passes=False
        ),
        scratch_types=[
            pltpu.VMEM((S, 128), jnp.int32),
            pltpu.VMEM((NB, 128, D), jnp.float32),
            pltpu.VMEM((NB, D8, 8, 128), jnp.float32),
            pltpu.SemaphoreType.DMA((NB,)),
            pltpu.SemaphoreType.DMA((NB,)),
        ],
    )
    def emb(tok_hbm, table_hbm, regs_hbm, out_hbm, idx_all, emb_v, trn_v, gsem, tsem):
        wid = lax.axis_index("s") * NUM_CORES + lax.axis_index("c")
        c = wid
        pltpu.sync_copy(tok_hbm.at[:, pl.ds(c * 128, 128)], idx_all)
        iota = lax.iota(jnp.int32, 16)
        perm = [jnp.bitwise_and(iota + o, 15) for o in range(16)]

        def gather_copy(slot, s):
            return pltpu.make_async_copy(
                table_hbm.at[idx_all.at[s]], emb_v.at[slot], gsem.at[slot]
            )

        def fire_out(slot, u):
            pltpu.async_copy(trn_v.at[slot], out_hbm.at[u, :, c], tsem.at[slot])

        def wait_out(slot, u):
            pltpu.make_async_copy(
                trn_v.at[slot], out_hbm.at[u, :, c], tsem.at[slot]
            ).wait()

        for t in range(R):
            if t >= NB:
                wait_out(t % NB, t - NB)
            pltpu.sync_copy(regs_hbm.at[t], trn_v.at[t % NB])
            fire_out(t % NB, t)

        for b in range(NB):
            gather_copy(b, b).start()

        def body(g, _):
            for b in range(NB):
                s = g * NB + b
                u = R + s
                gather_copy(b, s).wait()
                prev_done = lax.select(g > 0, u - NB, R - NB + b)
                wait_out(b, prev_done)

                slotv = jnp.full((16,), b, jnp.int32)

                @plsc.parallel_loop(0, (D // 16) * 8, unroll=1)
                def blk(m):
                    q = m // 8
                    k = m % 8
                    rowv = iota + 16 * k
                    colbase = 16 * q
                    for o in range(16):
                        colv = colbase + perm[o]
                        v = plsc.load_gather(emb_v, [slotv, rowv, colv])
                        plsc.store_scatter(
                            trn_v,
                            [
                                slotv,
                                jax.lax.shift_right_logical(colv, 3),
                                jnp.bitwise_and(colv, 7),
                                rowv,
                            ],
                            v,
                        )
                fire_out(b, u)

                @pl.when(g < NGRP - 1)
                def _():
                    gather_copy(b, s + NB).start()

            return ()

        lax.fori_loop(0, NGRP, body, ())

        for b in range(NB):
            wait_out(b, T - NB + b)

    out5 = emb(tokens_t, table, regs5)
    return jnp.transpose(out5, (2, 4, 0, 1, 3)).reshape(B, T, D)
